# Initial kernel scaffold; baseline (speedup 1.0000x reference)
import jax
import jax.numpy as jnp
from jax.experimental import pallas as pl


def kernel(x, edge_index, edge_attr, W1, b1, W2, b2, W3, b3):
    raise NotImplementedError("write your pallas kernel here")


# speedup vs baseline: 10.3136x; 10.3136x over previous
import functools

import jax
import jax.numpy as jnp
from jax import lax
from jax.experimental import pallas as pl
from jax.experimental.pallas import tpu as pltpu
from jax.experimental.pallas import tpu_sc as plsc

NC = 2
NS = 16
L = 16
K = 128


def _sc_mesh():
    return plsc.VectorSubcoreMesh(core_axis_name="c", subcore_axis_name="s")


def _deg_kernel(NP, EP):
    T = EP // (NC * NS * K)
    rpt = NP // NS

    @functools.partial(
        pl.kernel, mesh=_sc_mesh(),
        out_type=jax.ShapeDtypeStruct((NC, NP), jnp.float32),
        scratch_types=[
            pltpu.VMEM((K,), jnp.int32),
            pltpu.VMEM((K,), jnp.float32),
            pltpu.VMEM((rpt,), jnp.float32),
            pltpu.VMEM_SHARED((NP,), jnp.float32),
        ],
    )
    def deg_k(col_hbm, ew_hbm, out_hbm, colbuf, ewbuf, zbuf, acc_sh):
        c = lax.axis_index("c")
        s = lax.axis_index("s")

        def zb(i, carry):
            zbuf[pl.ds(i * L, L)] = jnp.zeros((L,), jnp.float32)
            return carry
        lax.fori_loop(0, rpt // L, zb, 0)
        pltpu.sync_copy(zbuf, acc_sh.at[pl.ds(s * rpt, rpt)])
        plsc.subcore_barrier()

        base = (c * NS + s) * (T * K)

        def chunk(t, carry):
            off = base + t * K
            pltpu.sync_copy(col_hbm.at[pl.ds(off, K)], colbuf)
            pltpu.sync_copy(ew_hbm.at[pl.ds(off, K)], ewbuf)
            pltpu.sync_copy(ewbuf, acc_sh.at[colbuf], add=True)
            return carry
        lax.fori_loop(0, T, chunk, 0)
        plsc.subcore_barrier()
        pltpu.sync_copy(acc_sh.at[pl.ds(s * rpt, rpt)],
                        out_hbm.at[c, pl.ds(s * rpt, rpt)])

    return deg_k


def _agg_kernel(NP, EP, F):
    T = EP // (NC * NS * K)
    rpt = NP // NS
    ZC = rpt // K

    @functools.partial(
        pl.kernel, mesh=_sc_mesh(),
        out_type=jax.ShapeDtypeStruct((NC, NP, F), jnp.float32),
        scratch_types=[
            pltpu.VMEM((K,), jnp.int32),
            pltpu.VMEM((K,), jnp.int32),
            pltpu.VMEM((K,), jnp.float32),
            pltpu.VMEM((K, F), jnp.float32),
            pltpu.VMEM_SHARED((NP, F), jnp.float32),
            pltpu.SemaphoreType.DMA,
        ],
        compiler_params=pltpu.CompilerParams(
            needs_layout_passes=False, use_tc_tiling_on_sc=False),
    )
    def agg_k(row_hbm, col_hbm, ew_hbm, p_hbm, out_hbm,
              rowbuf, colbuf, ewbuf, rows, acc_sh, sem):
        c = lax.axis_index("c")
        s = lax.axis_index("s")

        def zb(i, carry):
            for j in range(F // L):
                rows[i, pl.ds(j * L, L)] = jnp.zeros((L,), jnp.float32)
            return carry
        lax.fori_loop(0, K, zb, 0)
        for z in range(ZC):
            pltpu.sync_copy(rows, acc_sh.at[pl.ds(s * rpt + z * K, K)])
        plsc.subcore_barrier()

        base = (c * NS + s) * (T * K)

        def chunk(t, carry):
            off = base + t * K
            pltpu.sync_copy(row_hbm.at[pl.ds(off, K)], rowbuf)
            pltpu.sync_copy(col_hbm.at[pl.ds(off, K)], colbuf)
            pltpu.sync_copy(ew_hbm.at[pl.ds(off, K)], ewbuf)
            pltpu.async_copy(p_hbm.at[rowbuf], rows, sem).wait()

            def scale(i, carry2):
                w = plsc.load_gather(ewbuf, [jnp.full((L,), i, jnp.int32)])
                for j in range(F // L):
                    sl = (i, pl.ds(j * L, L))
                    rows[sl] = rows[sl] * w
                return carry2
            lax.fori_loop(0, K, scale, 0)
            pltpu.sync_copy(rows, acc_sh.at[colbuf], add=True)
            return carry
        lax.fori_loop(0, T, chunk, 0)
        plsc.subcore_barrier()
        for z in range(ZC):
            sl = pl.ds(s * rpt + z * K, K)
            pltpu.sync_copy(acc_sh.at[sl], out_hbm.at[c, sl])

    return agg_k


def _tc_norm_first(NP, D, H1, BLK):
    grid = NP // BLK

    def body(dega, degb, x_ref, w_ref, dinv_ref, g_ref, p_ref):
        d = dega[...] + degb[...] + 2.0
        di = jnp.where(d > 0, lax.rsqrt(d), 0.0)
        g = jnp.dot(x_ref[...], w_ref[...], preferred_element_type=jnp.float32)
        dinv_ref[...] = di
        g_ref[...] = g
        p_ref[...] = di * g

    return pl.pallas_call(
        body,
        grid=(grid,),
        in_specs=[
            pl.BlockSpec((BLK, 1), lambda i: (i, 0)),
            pl.BlockSpec((BLK, 1), lambda i: (i, 0)),
            pl.BlockSpec((BLK, D), lambda i: (i, 0)),
            pl.BlockSpec((D, H1), lambda i: (0, 0)),
        ],
        out_specs=[
            pl.BlockSpec((BLK, 1), lambda i: (i, 0)),
            pl.BlockSpec((BLK, H1), lambda i: (i, 0)),
            pl.BlockSpec((BLK, H1), lambda i: (i, 0)),
        ],
        out_shape=[
            jax.ShapeDtypeStruct((NP, 1), jnp.float32),
            jax.ShapeDtypeStruct((NP, H1), jnp.float32),
            jax.ShapeDtypeStruct((NP, H1), jnp.float32),
        ],
    )


def _tc_mid(NP, Fin, Fout, BLK):
    grid = NP // BLK

    def body(acca, accb, g_ref, dinv_ref, b_ref, w_ref, gout_ref, pout_ref):
        di = dinv_ref[...]
        a = di * (acca[...] + accb[...]) + (2.0 * di * di) * g_ref[...] + b_ref[...]
        h = jnp.maximum(a, 0.0)
        gn = jnp.dot(h, w_ref[...], preferred_element_type=jnp.float32)
        gout_ref[...] = gn
        pout_ref[...] = di * gn

    return pl.pallas_call(
        body,
        grid=(grid,),
        in_specs=[
            pl.BlockSpec((BLK, Fin), lambda i: (i, 0)),
            pl.BlockSpec((BLK, Fin), lambda i: (i, 0)),
            pl.BlockSpec((BLK, Fin), lambda i: (i, 0)),
            pl.BlockSpec((BLK, 1), lambda i: (i, 0)),
            pl.BlockSpec((1, Fin), lambda i: (0, 0)),
            pl.BlockSpec((Fin, Fout), lambda i: (0, 0)),
        ],
        out_specs=[
            pl.BlockSpec((BLK, Fout), lambda i: (i, 0)),
            pl.BlockSpec((BLK, Fout), lambda i: (i, 0)),
        ],
        out_shape=[
            jax.ShapeDtypeStruct((NP, Fout), jnp.float32),
            jax.ShapeDtypeStruct((NP, Fout), jnp.float32),
        ],
    )


def _tc_final(NP, F3, C, BLK):
    grid = NP // BLK

    def body(acca, accb, g_ref, dinv_ref, b_ref, out_ref):
        di = dinv_ref[...]
        a = di * (acca[...] + accb[...]) + (2.0 * di * di) * g_ref[...] + b_ref[...]
        mask = lax.broadcasted_iota(jnp.int32, (BLK, F3), 1) < C
        am = jnp.where(mask, a, -1e30)
        m = jnp.max(am, axis=1, keepdims=True)
        z = am - m
        e = jnp.where(mask, jnp.exp(z), 0.0)
        ssum = jnp.sum(e, axis=1, keepdims=True)
        out_ref[...] = z - jnp.log(ssum)

    return pl.pallas_call(
        body,
        grid=(grid,),
        in_specs=[
            pl.BlockSpec((BLK, F3), lambda i: (i, 0)),
            pl.BlockSpec((BLK, F3), lambda i: (i, 0)),
            pl.BlockSpec((BLK, F3), lambda i: (i, 0)),
            pl.BlockSpec((BLK, 1), lambda i: (i, 0)),
            pl.BlockSpec((1, F3), lambda i: (0, 0)),
        ],
        out_specs=pl.BlockSpec((BLK, F3), lambda i: (i, 0)),
        out_shape=jax.ShapeDtypeStruct((NP, F3), jnp.float32),
    )


def kernel(x, edge_index, edge_attr, W1, b1, W2, b2, W3, b3):
    N, D = x.shape
    E = edge_index.shape[1]
    H1 = W1.shape[1]
    H2 = W2.shape[1]
    C = W3.shape[1]
    F3 = 16
    NP = -(-N // 2048) * 2048
    EP = -(-E // (NC * NS * K)) * (NC * NS * K)
    BLK = 1024

    row = edge_index[0]
    col = edge_index[1]
    pad = EP - E
    if pad:
        pidx = (jnp.arange(pad, dtype=jnp.int32) % N).astype(jnp.int32)
        row = jnp.concatenate([row, pidx])
        col = jnp.concatenate([col, pidx])
        ew = jnp.concatenate([edge_attr, jnp.zeros((pad,), edge_attr.dtype)])
    else:
        ew = edge_attr
    xp = jnp.pad(x, ((0, NP - N), (0, 0)))
    W3p = jnp.pad(W3, ((0, 0), (0, F3 - C)))
    b1r = b1[None, :]
    b2r = b2[None, :]
    b3r = jnp.pad(b3, (0, F3 - C))[None, :]

    deg = _deg_kernel(NP, EP)(col, ew)
    dega = deg[0][:, None]
    degb = deg[1][:, None]
    dinv, g1, p1 = _tc_norm_first(NP, D, H1, BLK)(dega, degb, xp, W1)
    acc1 = _agg_kernel(NP, EP, H1)(row, col, ew, p1)
    g2, p2 = _tc_mid(NP, H1, H2, BLK)(acc1[0], acc1[1], g1, dinv, b1r, W2)
    acc2 = _agg_kernel(NP, EP, H2)(row, col, ew, p2)
    g3, p3 = _tc_mid(NP, H2, F3, BLK)(acc2[0], acc2[1], g2, dinv, b2r, W3p)
    acc3 = _agg_kernel(NP, EP, F3)(row, col, ew, p3)
    out = _tc_final(NP, F3, C, BLK)(acc3[0], acc3[1], g3, dinv, b3r)
    return out[:N, :C]


# speedup vs baseline: 22.7851x; 2.2092x over previous
import functools

import jax
import jax.numpy as jnp
from jax import lax
from jax.experimental import pallas as pl
from jax.experimental.pallas import tpu as pltpu
from jax.experimental.pallas import tpu_sc as plsc

NC = 2
NS = 16
NW = NC * NS
L = 16
K = 128

_SC_PARAMS = pltpu.CompilerParams(
    needs_layout_passes=False, use_tc_tiling_on_sc=False)


def _sc_mesh():
    return plsc.VectorSubcoreMesh(core_axis_name="c", subcore_axis_name="s")


def _deg_kernel(NP, EP):
    T = EP // (NW * K)
    rpt = NP // NS

    @functools.partial(
        pl.kernel, mesh=_sc_mesh(),
        out_type=jax.ShapeDtypeStruct((NC, NP), jnp.float32),
        scratch_types=[
            pltpu.VMEM((T, K), jnp.int32),
            pltpu.VMEM((T, K), jnp.float32),
            pltpu.VMEM((rpt,), jnp.float32),
            pltpu.VMEM_SHARED((NP,), jnp.float32),
        ],
        compiler_params=_SC_PARAMS,
    )
    def deg_k(col_hbm, ew_hbm, out_hbm, col_all, ew_all, zbuf, acc_sh):
        c = lax.axis_index("c")
        s = lax.axis_index("s")
        w = c * NS + s
        pltpu.sync_copy(col_hbm.at[w], col_all)
        pltpu.sync_copy(ew_hbm.at[w], ew_all)

        def zb(i, carry):
            zbuf[pl.ds(i * L, L)] = jnp.zeros((L,), jnp.float32)
            return carry
        lax.fori_loop(0, rpt // L, zb, 0)
        pltpu.sync_copy(zbuf, acc_sh.at[pl.ds(s * rpt, rpt)])
        plsc.subcore_barrier()

        def chunk(t, carry):
            pltpu.sync_copy(ew_all.at[t], acc_sh.at[col_all.at[t]], add=True)
            return carry
        lax.fori_loop(0, T, chunk, 0)
        plsc.subcore_barrier()
        pltpu.sync_copy(acc_sh.at[pl.ds(s * rpt, rpt)],
                        out_hbm.at[c, pl.ds(s * rpt, rpt)])

    return deg_k


def _pick_piece(T, F, NP):
    budget = 2_000_000
    best = 2
    for P in range(T, 1, -2):
        if T % P:
            continue
        words = NS * (3 * P * K + 2 * K * F) + NP * F
        if words <= budget:
            best = P
            break
    return best


def _agg_kernel(NP, EP, F):
    T = EP // (NW * K)
    P = _pick_piece(T, F, NP)
    QP = T // P
    P2 = P // 2
    rpt = NP // NS
    ZC = rpt // K

    @functools.partial(
        pl.kernel, mesh=_sc_mesh(),
        out_type=jax.ShapeDtypeStruct((NC, NP, F), jnp.float32),
        scratch_types=[
            pltpu.VMEM((P, K), jnp.int32),
            pltpu.VMEM((P, K), jnp.int32),
            pltpu.VMEM((P, K), jnp.float32),
            pltpu.VMEM((K, F), jnp.float32),
            pltpu.VMEM((K, F), jnp.float32),
            pltpu.VMEM_SHARED((NP, F), jnp.float32),
            pltpu.SemaphoreType.DMA,
            pltpu.SemaphoreType.DMA,
        ],
        compiler_params=_SC_PARAMS,
    )
    def agg_k(row_hbm, col_hbm, ew_hbm, p_hbm, out_hbm,
              row_all, col_all, ew_all, rows0, rows1, acc_sh, gsem0, gsem1):
        c = lax.axis_index("c")
        s = lax.axis_index("s")
        w = c * NS + s

        def zb(i, carry):
            for j in range(F // L):
                rows0[i, pl.ds(j * L, L)] = jnp.zeros((L,), jnp.float32)
            return carry
        lax.fori_loop(0, K, zb, 0)
        for z in range(ZC):
            pltpu.sync_copy(rows0, acc_sh.at[pl.ds(s * rpt + z * K, K)])
        plsc.subcore_barrier()

        def scale(rbuf, t):
            tv = jnp.full((L,), t, jnp.int32)

            def sbody(i, carry2):
                wv = plsc.load_gather(
                    ew_all, [tv, jnp.full((L,), i, jnp.int32)])
                for j in range(F // L):
                    sl = (i, pl.ds(j * L, L))
                    rbuf[sl] = rbuf[sl] * wv
                return carry2
            lax.fori_loop(0, K, sbody, 0, unroll=4)

        def gstart(rbuf, t, sem):
            pltpu.async_copy(p_hbm.at[row_all.at[t]], rbuf, sem)

        def gwait(rbuf, t, sem):
            pltpu.make_async_copy(p_hbm.at[row_all.at[t]], rbuf, sem).wait()

        def piece(q, carry):
            pltpu.sync_copy(row_hbm.at[w, pl.ds(q * P, P)], row_all)
            pltpu.sync_copy(col_hbm.at[w, pl.ds(q * P, P)], col_all)
            pltpu.sync_copy(ew_hbm.at[w, pl.ds(q * P, P)], ew_all)
            gstart(rows0, 0, gsem0)

            def pair(k2, carry2):
                u = 2 * k2
                gstart(rows1, u + 1, gsem1)
                gwait(rows0, u, gsem0)
                scale(rows0, u)
                pltpu.sync_copy(rows0, acc_sh.at[col_all.at[u]], add=True)
                gstart(rows0, lax.rem(u + 2, P), gsem0)
                gwait(rows1, u + 1, gsem1)
                scale(rows1, u + 1)
                pltpu.sync_copy(rows1, acc_sh.at[col_all.at[u + 1]], add=True)
                return carry2
            lax.fori_loop(0, P2, pair, 0)
            gwait(rows0, 0, gsem0)
            return carry
        lax.fori_loop(0, QP, piece, 0)
        plsc.subcore_barrier()
        for z in range(ZC):
            sl = pl.ds(s * rpt + z * K, K)
            pltpu.sync_copy(acc_sh.at[sl], out_hbm.at[c, sl])

    return agg_k


def _tc_norm_first(NP, D, H1, BLK):
    grid = NP // BLK

    def body(dega, degb, x_ref, w_ref, dinv_ref, g_ref, p_ref):
        d = dega[...] + degb[...] + 2.0
        di = jnp.where(d > 0, lax.rsqrt(d), 0.0)
        g = jnp.dot(x_ref[...], w_ref[...], preferred_element_type=jnp.float32)
        dinv_ref[...] = di
        g_ref[...] = g
        p_ref[...] = di * g

    return pl.pallas_call(
        body,
        grid=(grid,),
        in_specs=[
            pl.BlockSpec((BLK, 1), lambda i: (i, 0)),
            pl.BlockSpec((BLK, 1), lambda i: (i, 0)),
            pl.BlockSpec((BLK, D), lambda i: (i, 0)),
            pl.BlockSpec((D, H1), lambda i: (0, 0)),
        ],
        out_specs=[
            pl.BlockSpec((BLK, 1), lambda i: (i, 0)),
            pl.BlockSpec((BLK, H1), lambda i: (i, 0)),
            pl.BlockSpec((BLK, H1), lambda i: (i, 0)),
        ],
        out_shape=[
            jax.ShapeDtypeStruct((NP, 1), jnp.float32),
            jax.ShapeDtypeStruct((NP, H1), jnp.float32),
            jax.ShapeDtypeStruct((NP, H1), jnp.float32),
        ],
    )


def _tc_mid(NP, Fin, Fout, BLK):
    grid = NP // BLK

    def body(acca, accb, g_ref, dinv_ref, b_ref, w_ref, gout_ref, pout_ref):
        di = dinv_ref[...]
        a = di * (acca[...] + accb[...]) + (2.0 * di * di) * g_ref[...] + b_ref[...]
        h = jnp.maximum(a, 0.0)
        gn = jnp.dot(h, w_ref[...], preferred_element_type=jnp.float32)
        gout_ref[...] = gn
        pout_ref[...] = di * gn

    return pl.pallas_call(
        body,
        grid=(grid,),
        in_specs=[
            pl.BlockSpec((BLK, Fin), lambda i: (i, 0)),
            pl.BlockSpec((BLK, Fin), lambda i: (i, 0)),
            pl.BlockSpec((BLK, Fin), lambda i: (i, 0)),
            pl.BlockSpec((BLK, 1), lambda i: (i, 0)),
            pl.BlockSpec((1, Fin), lambda i: (0, 0)),
            pl.BlockSpec((Fin, Fout), lambda i: (0, 0)),
        ],
        out_specs=[
            pl.BlockSpec((BLK, Fout), lambda i: (i, 0)),
            pl.BlockSpec((BLK, Fout), lambda i: (i, 0)),
        ],
        out_shape=[
            jax.ShapeDtypeStruct((NP, Fout), jnp.float32),
            jax.ShapeDtypeStruct((NP, Fout), jnp.float32),
        ],
    )


def _tc_final(NP, F3, C, BLK):
    grid = NP // BLK

    def body(acca, accb, g_ref, dinv_ref, b_ref, out_ref):
        di = dinv_ref[...]
        a = di * (acca[...] + accb[...]) + (2.0 * di * di) * g_ref[...] + b_ref[...]
        mask = lax.broadcasted_iota(jnp.int32, (BLK, F3), 1) < C
        am = jnp.where(mask, a, -1e30)
        m = jnp.max(am, axis=1, keepdims=True)
        z = am - m
        e = jnp.where(mask, jnp.exp(z), 0.0)
        ssum = jnp.sum(e, axis=1, keepdims=True)
        out_ref[...] = z - jnp.log(ssum)

    return pl.pallas_call(
        body,
        grid=(grid,),
        in_specs=[
            pl.BlockSpec((BLK, F3), lambda i: (i, 0)),
            pl.BlockSpec((BLK, F3), lambda i: (i, 0)),
            pl.BlockSpec((BLK, F3), lambda i: (i, 0)),
            pl.BlockSpec((BLK, 1), lambda i: (i, 0)),
            pl.BlockSpec((1, F3), lambda i: (0, 0)),
        ],
        out_specs=pl.BlockSpec((BLK, F3), lambda i: (i, 0)),
        out_shape=jax.ShapeDtypeStruct((NP, F3), jnp.float32),
    )


def kernel(x, edge_index, edge_attr, W1, b1, W2, b2, W3, b3):
    N, D = x.shape
    E = edge_index.shape[1]
    H1 = W1.shape[1]
    H2 = W2.shape[1]
    C = W3.shape[1]
    F3 = 16
    NP = -(-N // 2048) * 2048
    EP = -(-E // (NW * K * 2)) * (NW * K * 2)
    BLK = 1024

    row = edge_index[0]
    col = edge_index[1]
    pad = EP - E
    if pad:
        pidx = (jnp.arange(pad, dtype=jnp.int32) % N).astype(jnp.int32)
        row = jnp.concatenate([row, pidx])
        col = jnp.concatenate([col, pidx])
        ew = jnp.concatenate([edge_attr, jnp.zeros((pad,), edge_attr.dtype)])
    else:
        ew = edge_attr
    row3 = row.reshape(NW, -1, K)
    col3 = col.reshape(NW, -1, K)
    ew3 = ew.reshape(NW, -1, K)
    xp = jnp.pad(x, ((0, NP - N), (0, 0)))
    W3p = jnp.pad(W3, ((0, 0), (0, F3 - C)))
    b1r = b1[None, :]
    b2r = b2[None, :]
    b3r = jnp.pad(b3, (0, F3 - C))[None, :]

    deg = _deg_kernel(NP, EP)(col3, ew3)
    dega = deg[0][:, None]
    degb = deg[1][:, None]
    dinv, g1, p1 = _tc_norm_first(NP, D, H1, BLK)(dega, degb, xp, W1)
    acc1 = _agg_kernel(NP, EP, H1)(row3, col3, ew3, p1)
    g2, p2 = _tc_mid(NP, H1, H2, BLK)(acc1[0], acc1[1], g1, dinv, b1r, W2)
    acc2 = _agg_kernel(NP, EP, H2)(row3, col3, ew3, p2)
    g3, p3 = _tc_mid(NP, H2, F3, BLK)(acc2[0], acc2[1], g2, dinv, b2r, W3p)
    acc3 = _agg_kernel(NP, EP, F3)(row3, col3, ew3, p3)
    out = _tc_final(NP, F3, C, BLK)(acc3[0], acc3[1], g3, dinv, b3r)
    return out[:N, :C]
